# baseline (speedup 1.0000x reference)
import functools

import jax
import jax.numpy as jnp
from jax import lax
from jax.experimental import pallas as pl
from jax.experimental.pallas import tpu as pltpu
from jax.experimental.pallas import tpu_sc as plsc

NBLK = 2
DIM = 1024
QH = 16
KVH = 4
NH = QH + 2 * KVH
HD = 64
BATCH = 2
SEQ = 2048
CAP = 1024
CHUNK = 256
NCH = SEQ // CHUNK
SCALE = 1.0 / (HD ** 0.5)
EPS = 1e-6

_PREC = lax.Precision.HIGHEST

SC_NC = 2
SC_NS = 16
SC_NW = SC_NC * SC_NS
G_ROWS = (BATCH * CAP) // SC_NW


def _select_body(vrow_ref, vcol_ref, dec_ref, gidx_ref, anti_ref, mult_ref):
    v_row = vrow_ref[0]
    v_col = vcol_ref[0]

    rank_row = jnp.zeros((1, SEQ), jnp.int32)
    for c in range(NCH):
        vs = v_col[c * CHUNK:(c + 1) * CHUNK, :]
        s_idx = lax.broadcasted_iota(jnp.int32, (CHUNK, SEQ), 0) + c * CHUNK
        t_idx = lax.broadcasted_iota(jnp.int32, (CHUNK, SEQ), 1)
        beat = (vs > v_row) | ((vs == v_row) & (s_idx < t_idx))
        rank_row += jnp.sum(beat.astype(jnp.int32), axis=0, keepdims=True)
    sel_row = rank_row < CAP
    dec_ref[0] = sel_row.astype(jnp.float32)

    rank_col = jnp.zeros((SEQ, 1), jnp.int32)
    for c in range(NCH):
        vt = v_row[:, c * CHUNK:(c + 1) * CHUNK]
        t_idx = lax.broadcasted_iota(jnp.int32, (SEQ, CHUNK), 1) + c * CHUNK
        s_idx = lax.broadcasted_iota(jnp.int32, (SEQ, CHUNK), 0)
        beat = (vt > v_col) | ((vt == v_col) & (t_idx < s_idx))
        rank_col += jnp.sum(beat.astype(jnp.int32), axis=1, keepdims=True)
    sel_col = rank_col < CAP

    pos_row = jnp.zeros((1, SEQ), jnp.int32)
    for c in range(NCH):
        selc = sel_col[c * CHUNK:(c + 1) * CHUNK, :]
        s_idx = lax.broadcasted_iota(jnp.int32, (CHUNK, SEQ), 0) + c * CHUNK
        t_idx = lax.broadcasted_iota(jnp.int32, (CHUNK, SEQ), 1)
        pos_row += jnp.sum((selc & (s_idx < t_idx)).astype(jnp.int32),
                           axis=0, keepdims=True)

    w_row = jnp.zeros((1, SEQ), jnp.float32)
    for c in range(NCH):
        vr = v_col[c * CHUNK:(c + 1) * CHUNK, :]
        r_idx = lax.broadcasted_iota(jnp.int32, (CHUNK, SEQ), 0) + c * CHUNK
        w_row += jnp.sum(jnp.where(r_idx == rank_row, vr, 0.0),
                         axis=0, keepdims=True)

    jv = lax.broadcasted_iota(jnp.int32, (CAP, 1), 0)
    sidx_acc = jnp.zeros((CAP, 1), jnp.int32)
    anti_acc = jnp.zeros((CAP, 1), jnp.int32)
    mult_acc = jnp.zeros((CAP, 1), jnp.float32)
    for c in range(NCH):
        selr = sel_row[:, c * CHUNK:(c + 1) * CHUNK]
        posr = pos_row[:, c * CHUNK:(c + 1) * CHUNK]
        wr = w_row[:, c * CHUNK:(c + 1) * CHUNK]
        s_iota = lax.broadcasted_iota(jnp.int32, (1, CHUNK), 1) + c * CHUNK
        m2 = selr & (posr == jv)
        sidx_acc += jnp.sum(jnp.where(m2, s_iota, 0), axis=1, keepdims=True)
        mult_acc += jnp.sum(jnp.where(m2, wr, 0.0), axis=1, keepdims=True)
        antipos = s_iota - posr
        m3 = (~selr) & (antipos == jv)
        anti_acc += jnp.sum(jnp.where(m3, s_iota, 0), axis=1, keepdims=True)

    b = pl.program_id(0)
    gidx_ref[0] = sidx_acc + b * SEQ
    anti_ref[0] = anti_acc + b * SEQ
    mult_ref[0] = mult_acc


def _select_call(rw_rows3, rw_cols3):
    return pl.pallas_call(
        _select_body,
        grid=(BATCH,),
        in_specs=[
            pl.BlockSpec((1, 1, SEQ), lambda b: (b, 0, 0)),
            pl.BlockSpec((1, SEQ, 1), lambda b: (b, 0, 0)),
        ],
        out_specs=[
            pl.BlockSpec((1, 1, SEQ), lambda b: (b, 0, 0)),
            pl.BlockSpec((1, CAP, 1), lambda b: (b, 0, 0)),
            pl.BlockSpec((1, CAP, 1), lambda b: (b, 0, 0)),
            pl.BlockSpec((1, CAP, 1), lambda b: (b, 0, 0)),
        ],
        out_shape=[
            jax.ShapeDtypeStruct((BATCH, 1, SEQ), jnp.float32),
            jax.ShapeDtypeStruct((BATCH, CAP, 1), jnp.int32),
            jax.ShapeDtypeStruct((BATCH, CAP, 1), jnp.int32),
            jax.ShapeDtypeStruct((BATCH, CAP, 1), jnp.float32),
        ],
    )(rw_rows3, rw_cols3)


def _qkv_body(x_ref, n_ref, w_ref, out_ref):
    xb = x_ref[0]
    var = jnp.mean(xb * xb, axis=1, keepdims=True)
    h = xb * lax.rsqrt(var + EPS) * n_ref[...]
    res = jnp.dot(h, w_ref[...], preferred_element_type=jnp.float32,
                  precision=_PREC)
    for hh in range(NH):
        out_ref[0, hh] = res[:, hh * HD:(hh + 1) * HD]


def _qkv_call(latent, n1, wqkv_t):
    return pl.pallas_call(
        _qkv_body,
        grid=(BATCH,),
        in_specs=[
            pl.BlockSpec((1, CAP, DIM), lambda b: (b, 0, 0)),
            pl.BlockSpec((1, DIM), lambda b: (0, 0)),
            pl.BlockSpec((DIM, NH * HD), lambda b: (0, 0)),
        ],
        out_specs=pl.BlockSpec((1, NH, CAP, HD), lambda b: (b, 0, 0, 0)),
        out_shape=jax.ShapeDtypeStruct((BATCH, NH, CAP, HD), jnp.float32),
    )(latent, n1, wqkv_t)


def _att_body(q_ref, k_ref, v_ref, e_ref, slopes_ref, out_ref):
    q = q_ref[0, 0]
    k = k_ref[0, 0]
    v = v_ref[0, 0]
    h = pl.program_id(1)
    slope = slopes_ref[h]
    scores = lax.dot_general(q, k, (((1,), (1,)), ((), ())),
                             preferred_element_type=jnp.float32,
                             precision=_PREC)
    s2 = scores * SCALE + slope * e_ref[...]
    m = jnp.max(s2, axis=1, keepdims=True)
    p = jnp.exp(s2 - m)
    l = jnp.sum(p, axis=1, keepdims=True)
    o = jnp.dot(p, v, preferred_element_type=jnp.float32, precision=_PREC)
    out_ref[0, 0] = o / l


def _att_call(qkv, e_mat, slopes):
    return pl.pallas_call(
        _att_body,
        grid=(BATCH, QH),
        in_specs=[
            pl.BlockSpec((1, 1, CAP, HD), lambda b, h: (b, h, 0, 0)),
            pl.BlockSpec((1, 1, CAP, HD), lambda b, h: (b, QH + h // 4, 0, 0)),
            pl.BlockSpec((1, 1, CAP, HD),
                         lambda b, h: (b, QH + KVH + h // 4, 0, 0)),
            pl.BlockSpec((CAP, CAP), lambda b, h: (0, 0)),
            pl.BlockSpec(memory_space=pltpu.SMEM),
        ],
        out_specs=pl.BlockSpec((1, 1, CAP, HD), lambda b, h: (b, h, 0, 0)),
        out_shape=jax.ShapeDtypeStruct((BATCH, QH, CAP, HD), jnp.float32),
    )(qkv, qkv, qkv, e_mat, slopes)


def _proj_body(a_ref, w_ref, x_ref, out_ref):
    a4 = a_ref[0]
    a2 = jnp.concatenate([a4[hh] for hh in range(QH)], axis=1)
    out_ref[0] = x_ref[0] + jnp.dot(a2, w_ref[...],
                                    preferred_element_type=jnp.float32,
                                    precision=_PREC)


def _proj_call(attn, wproj_t, latent):
    return pl.pallas_call(
        _proj_body,
        grid=(BATCH,),
        in_specs=[
            pl.BlockSpec((1, QH, CAP, HD), lambda b: (b, 0, 0, 0)),
            pl.BlockSpec((DIM, DIM), lambda b: (0, 0)),
            pl.BlockSpec((1, CAP, DIM), lambda b: (b, 0, 0)),
        ],
        out_specs=pl.BlockSpec((1, CAP, DIM), lambda b: (b, 0, 0)),
        out_shape=jax.ShapeDtypeStruct((BATCH, CAP, DIM), jnp.float32),
    )(attn, wproj_t, latent)


def _mlp_common(x_ref, n_ref, w1_ref, w2_ref):
    xb = x_ref[0]
    var = jnp.mean(xb * xb, axis=1, keepdims=True)
    h = xb * lax.rsqrt(var + EPS) * n_ref[...]
    hh = jnp.dot(h, w1_ref[...], preferred_element_type=jnp.float32,
                 precision=_PREC)
    x1 = hh[:, :DIM]
    gate = hh[:, DIM:]
    act = x1 * (gate * (1.0 / (1.0 + jnp.exp(-gate))))
    y = jnp.dot(act, w2_ref[...], preferred_element_type=jnp.float32,
                precision=_PREC)
    return xb + y


def _mlp_body(x_ref, n_ref, w1_ref, w2_ref, out_ref):
    out_ref[0] = _mlp_common(x_ref, n_ref, w1_ref, w2_ref)


def _mlp_final_body(x_ref, n_ref, w1_ref, w2_ref, mult_ref, out_ref):
    out_ref[0] = _mlp_common(x_ref, n_ref, w1_ref, w2_ref) * mult_ref[0]


def _mlp_call(latent, n2, w1_t, w2_t, mult=None):
    in_specs = [
        pl.BlockSpec((1, CAP, DIM), lambda b: (b, 0, 0)),
        pl.BlockSpec((1, DIM), lambda b: (0, 0)),
        pl.BlockSpec((DIM, 2 * DIM), lambda b: (0, 0)),
        pl.BlockSpec((DIM, DIM), lambda b: (0, 0)),
    ]
    args = [latent, n2, w1_t, w2_t]
    body = _mlp_body
    if mult is not None:
        in_specs.append(pl.BlockSpec((1, CAP, 1), lambda b: (b, 0, 0)))
        args.append(mult)
        body = _mlp_final_body
    return pl.pallas_call(
        body,
        grid=(BATCH,),
        in_specs=in_specs,
        out_specs=pl.BlockSpec((1, CAP, DIM), lambda b: (b, 0, 0)),
        out_shape=jax.ShapeDtypeStruct((BATCH, CAP, DIM), jnp.float32),
    )(*args)


@functools.cache
def _sc_mesh():
    return plsc.VectorSubcoreMesh(core_axis_name="c", subcore_axis_name="s",
                                  num_cores=SC_NC, num_subcores=SC_NS)


@functools.cache
def _sc_gather_kernel():
    def body(x_hbm, idx_hbm, out_hbm, idx_v, rows_v, sem):
        wid = lax.axis_index("s") * SC_NC + lax.axis_index("c")
        base = wid * G_ROWS
        pltpu.sync_copy(idx_hbm.at[pl.ds(base, G_ROWS)], idx_v)
        pltpu.async_copy(x_hbm.at[idx_v], rows_v, sem).wait()
        pltpu.sync_copy(rows_v, out_hbm.at[pl.ds(base, G_ROWS)])

    return pl.kernel(
        body,
        out_type=jax.ShapeDtypeStruct((BATCH * CAP, DIM), jnp.float32),
        mesh=_sc_mesh(),
        scratch_types=[
            pltpu.VMEM((G_ROWS,), jnp.int32),
            pltpu.VMEM((G_ROWS, DIM), jnp.float32),
            pltpu.SemaphoreType.DMA,
        ],
    )


def _sc_gather(x2d, gidx):
    return _sc_gather_kernel()(x2d, gidx)


@functools.cache
def _sc_scatter_kernel():
    def body(lat_hbm, gidx_hbm, anti_hbm, zero_hbm, out_hbm,
             idx_v, rows_v, sem):
        wid = lax.axis_index("s") * SC_NC + lax.axis_index("c")
        base = wid * G_ROWS
        pltpu.sync_copy(gidx_hbm.at[pl.ds(base, G_ROWS)], idx_v)
        pltpu.sync_copy(lat_hbm.at[pl.ds(base, G_ROWS)], rows_v)
        pltpu.async_copy(rows_v, out_hbm.at[idx_v], sem).wait()
        pltpu.sync_copy(anti_hbm.at[pl.ds(base, G_ROWS)], idx_v)
        pltpu.sync_copy(zero_hbm, rows_v)
        pltpu.async_copy(rows_v, out_hbm.at[idx_v], sem).wait()

    return pl.kernel(
        body,
        out_type=jax.ShapeDtypeStruct((BATCH * SEQ, DIM), jnp.float32),
        mesh=_sc_mesh(),
        scratch_types=[
            pltpu.VMEM((G_ROWS,), jnp.int32),
            pltpu.VMEM((G_ROWS, DIM), jnp.float32),
            pltpu.SemaphoreType.DMA,
        ],
    )


def _sc_scatter(lat, gidx, anti, zero_rows):
    return _sc_scatter_kernel()(lat, gidx, anti, zero_rows)


def kernel(x, norm1_w, norm2_w, W_qkv, W_proj, W_fc1, W_fc2, W_router):
    f32 = jnp.float32

    i = jnp.arange(CAP, dtype=jnp.int32)[:, None]
    j = jnp.arange(CAP, dtype=jnp.int32)[None, :]
    e_mat = jnp.where(j <= i, (j - i).astype(f32), jnp.float32(-1e30))
    slopes = jnp.exp2(-((jnp.arange(QH, dtype=f32) + 1) * 8.0 / QH))

    rw_cols3 = jax.nn.sigmoid(jnp.einsum('bsd,od->bso', x, W_router))
    rw_rows3 = rw_cols3.reshape(BATCH, 1, SEQ)
    dec3, gidx3, anti3, mult3 = _select_call(rw_rows3, rw_cols3)
    gidx = gidx3.reshape(BATCH * CAP)
    anti = anti3.reshape(BATCH * CAP)

    x2d = x.reshape(BATCH * SEQ, DIM)
    latent = _sc_gather(x2d, gidx).reshape(BATCH, CAP, DIM)

    for blk in range(NBLK):
        wqkv_t = jnp.transpose(W_qkv[blk])
        wproj_t = jnp.transpose(W_proj[blk])
        w1_t = jnp.transpose(W_fc1[blk])
        w2_t = jnp.transpose(W_fc2[blk])
        n1 = norm1_w[blk].reshape(1, DIM)
        n2 = norm2_w[blk].reshape(1, DIM)
        qkv = _qkv_call(latent, n1, wqkv_t)
        attn = _att_call(qkv, e_mat, slopes)
        latent = _proj_call(attn, wproj_t, latent)
        latent = _mlp_call(latent, n2, w1_t, w2_t,
                           mult=mult3 if blk == NBLK - 1 else None)

    zero_rows = jnp.zeros((G_ROWS, DIM), f32)
    pred = _sc_scatter(latent.reshape(BATCH * CAP, DIM), gidx, anti,
                       zero_rows)
    pred = pred.reshape(BATCH, SEQ, DIM)

    return pred, rw_cols3, dec3.reshape(BATCH, SEQ, 1)


# speedup vs baseline: 3.0485x; 3.0485x over previous
import functools

import jax
import jax.numpy as jnp
from jax import lax
from jax.experimental import pallas as pl
from jax.experimental.pallas import tpu as pltpu
from jax.experimental.pallas import tpu_sc as plsc

NBLK = 2
DIM = 1024
QH = 16
KVH = 4
NH = QH + 2 * KVH
HD = 64
BATCH = 2
SEQ = 2048
CAP = 1024
CHUNK = 256
NCH = SEQ // CHUNK
SCALE = 1.0 / (HD ** 0.5)
EPS = 1e-6

_PREC = lax.Precision.DEFAULT

SC_NC = 2
SC_NS = 16
SC_NW = SC_NC * SC_NS
G_ROWS = (BATCH * CAP) // SC_NW


def _select_body(vrow_ref, vcol_ref, dec_ref, gidx_ref, anti_ref, mult_ref):
    v_row = vrow_ref[0]
    v_col = vcol_ref[0]

    rank_row = jnp.zeros((1, SEQ), jnp.int32)
    for c in range(NCH):
        vs = v_col[c * CHUNK:(c + 1) * CHUNK, :]
        s_idx = lax.broadcasted_iota(jnp.int32, (CHUNK, SEQ), 0) + c * CHUNK
        t_idx = lax.broadcasted_iota(jnp.int32, (CHUNK, SEQ), 1)
        beat = (vs > v_row) | ((vs == v_row) & (s_idx < t_idx))
        rank_row += jnp.sum(beat.astype(jnp.int32), axis=0, keepdims=True)
    sel_row = rank_row < CAP
    dec_ref[0] = sel_row.astype(jnp.float32)

    rank_col = jnp.zeros((SEQ, 1), jnp.int32)
    for c in range(NCH):
        vt = v_row[:, c * CHUNK:(c + 1) * CHUNK]
        t_idx = lax.broadcasted_iota(jnp.int32, (SEQ, CHUNK), 1) + c * CHUNK
        s_idx = lax.broadcasted_iota(jnp.int32, (SEQ, CHUNK), 0)
        beat = (vt > v_col) | ((vt == v_col) & (t_idx < s_idx))
        rank_col += jnp.sum(beat.astype(jnp.int32), axis=1, keepdims=True)
    sel_col = rank_col < CAP

    pos_row = jnp.zeros((1, SEQ), jnp.int32)
    for c in range(NCH):
        selc = sel_col[c * CHUNK:(c + 1) * CHUNK, :]
        s_idx = lax.broadcasted_iota(jnp.int32, (CHUNK, SEQ), 0) + c * CHUNK
        t_idx = lax.broadcasted_iota(jnp.int32, (CHUNK, SEQ), 1)
        pos_row += jnp.sum((selc & (s_idx < t_idx)).astype(jnp.int32),
                           axis=0, keepdims=True)

    w_row = jnp.zeros((1, SEQ), jnp.float32)
    for c in range(NCH):
        vr = v_col[c * CHUNK:(c + 1) * CHUNK, :]
        r_idx = lax.broadcasted_iota(jnp.int32, (CHUNK, SEQ), 0) + c * CHUNK
        w_row += jnp.sum(jnp.where(r_idx == rank_row, vr, 0.0),
                         axis=0, keepdims=True)

    jv = lax.broadcasted_iota(jnp.int32, (CAP, 1), 0)
    sidx_acc = jnp.zeros((CAP, 1), jnp.int32)
    anti_acc = jnp.zeros((CAP, 1), jnp.int32)
    mult_acc = jnp.zeros((CAP, 1), jnp.float32)
    for c in range(NCH):
        selr = sel_row[:, c * CHUNK:(c + 1) * CHUNK]
        posr = pos_row[:, c * CHUNK:(c + 1) * CHUNK]
        wr = w_row[:, c * CHUNK:(c + 1) * CHUNK]
        s_iota = lax.broadcasted_iota(jnp.int32, (1, CHUNK), 1) + c * CHUNK
        m2 = selr & (posr == jv)
        sidx_acc += jnp.sum(jnp.where(m2, s_iota, 0), axis=1, keepdims=True)
        mult_acc += jnp.sum(jnp.where(m2, wr, 0.0), axis=1, keepdims=True)
        antipos = s_iota - posr
        m3 = (~selr) & (antipos == jv)
        anti_acc += jnp.sum(jnp.where(m3, s_iota, 0), axis=1, keepdims=True)

    b = pl.program_id(0)
    gidx_ref[0] = sidx_acc + b * SEQ
    anti_ref[0] = anti_acc + b * SEQ
    mult_ref[0] = mult_acc


def _select_call(rw_rows3, rw_cols3):
    return pl.pallas_call(
        _select_body,
        grid=(BATCH,),
        in_specs=[
            pl.BlockSpec((1, 1, SEQ), lambda b: (b, 0, 0)),
            pl.BlockSpec((1, SEQ, 1), lambda b: (b, 0, 0)),
        ],
        out_specs=[
            pl.BlockSpec((1, 1, SEQ), lambda b: (b, 0, 0)),
            pl.BlockSpec((1, CAP, 1), lambda b: (b, 0, 0)),
            pl.BlockSpec((1, CAP, 1), lambda b: (b, 0, 0)),
            pl.BlockSpec((1, CAP, 1), lambda b: (b, 0, 0)),
        ],
        out_shape=[
            jax.ShapeDtypeStruct((BATCH, 1, SEQ), jnp.float32),
            jax.ShapeDtypeStruct((BATCH, CAP, 1), jnp.int32),
            jax.ShapeDtypeStruct((BATCH, CAP, 1), jnp.int32),
            jax.ShapeDtypeStruct((BATCH, CAP, 1), jnp.float32),
        ],
    )(rw_rows3, rw_cols3)


def _qkv_body(x_ref, n_ref, w_ref, out_ref):
    xb = x_ref[0]
    var = jnp.mean(xb * xb, axis=1, keepdims=True)
    h = xb * lax.rsqrt(var + EPS) * n_ref[...]
    res = jnp.dot(h, w_ref[...], preferred_element_type=jnp.float32,
                  precision=_PREC)
    for hh in range(NH):
        out_ref[0, hh] = res[:, hh * HD:(hh + 1) * HD]


def _qkv_call(latent, n1, wqkv_t):
    return pl.pallas_call(
        _qkv_body,
        grid=(BATCH,),
        in_specs=[
            pl.BlockSpec((1, CAP, DIM), lambda b: (b, 0, 0)),
            pl.BlockSpec((1, DIM), lambda b: (0, 0)),
            pl.BlockSpec((DIM, NH * HD), lambda b: (0, 0)),
        ],
        out_specs=pl.BlockSpec((1, NH, CAP, HD), lambda b: (b, 0, 0, 0)),
        out_shape=jax.ShapeDtypeStruct((BATCH, NH, CAP, HD), jnp.float32),
    )(latent, n1, wqkv_t)


def _att_body(q_ref, k_ref, v_ref, e_ref, slopes_ref, out_ref):
    q = q_ref[0, 0]
    k = k_ref[0, 0]
    v = v_ref[0, 0]
    h = pl.program_id(1)
    slope = slopes_ref[h]
    scores = lax.dot_general(q, k, (((1,), (1,)), ((), ())),
                             preferred_element_type=jnp.float32,
                             precision=_PREC)
    s2 = scores * SCALE + slope * e_ref[...]
    m = jnp.max(s2, axis=1, keepdims=True)
    p = jnp.exp(s2 - m)
    l = jnp.sum(p, axis=1, keepdims=True)
    o = jnp.dot(p, v, preferred_element_type=jnp.float32, precision=_PREC)
    out_ref[0, 0] = o / l


def _att_call(qkv, e_mat, slopes):
    return pl.pallas_call(
        _att_body,
        grid=(BATCH, QH),
        in_specs=[
            pl.BlockSpec((1, 1, CAP, HD), lambda b, h: (b, h, 0, 0)),
            pl.BlockSpec((1, 1, CAP, HD), lambda b, h: (b, QH + h // 4, 0, 0)),
            pl.BlockSpec((1, 1, CAP, HD),
                         lambda b, h: (b, QH + KVH + h // 4, 0, 0)),
            pl.BlockSpec((CAP, CAP), lambda b, h: (0, 0)),
            pl.BlockSpec(memory_space=pltpu.SMEM),
        ],
        out_specs=pl.BlockSpec((1, 1, CAP, HD), lambda b, h: (b, h, 0, 0)),
        out_shape=jax.ShapeDtypeStruct((BATCH, QH, CAP, HD), jnp.float32),
    )(qkv, qkv, qkv, e_mat, slopes)


def _proj_body(a_ref, w_ref, x_ref, out_ref):
    a4 = a_ref[0]
    a2 = jnp.concatenate([a4[hh] for hh in range(QH)], axis=1)
    out_ref[0] = x_ref[0] + jnp.dot(a2, w_ref[...],
                                    preferred_element_type=jnp.float32,
                                    precision=_PREC)


def _proj_call(attn, wproj_t, latent):
    return pl.pallas_call(
        _proj_body,
        grid=(BATCH,),
        in_specs=[
            pl.BlockSpec((1, QH, CAP, HD), lambda b: (b, 0, 0, 0)),
            pl.BlockSpec((DIM, DIM), lambda b: (0, 0)),
            pl.BlockSpec((1, CAP, DIM), lambda b: (b, 0, 0)),
        ],
        out_specs=pl.BlockSpec((1, CAP, DIM), lambda b: (b, 0, 0)),
        out_shape=jax.ShapeDtypeStruct((BATCH, CAP, DIM), jnp.float32),
    )(attn, wproj_t, latent)


def _mlp_common(x_ref, n_ref, w1_ref, w2_ref):
    xb = x_ref[0]
    var = jnp.mean(xb * xb, axis=1, keepdims=True)
    h = xb * lax.rsqrt(var + EPS) * n_ref[...]
    hh = jnp.dot(h, w1_ref[...], preferred_element_type=jnp.float32,
                 precision=_PREC)
    x1 = hh[:, :DIM]
    gate = hh[:, DIM:]
    act = x1 * (gate * (1.0 / (1.0 + jnp.exp(-gate))))
    y = jnp.dot(act, w2_ref[...], preferred_element_type=jnp.float32,
                precision=_PREC)
    return xb + y


def _mlp_body(x_ref, n_ref, w1_ref, w2_ref, out_ref):
    out_ref[0] = _mlp_common(x_ref, n_ref, w1_ref, w2_ref)


def _mlp_final_body(x_ref, n_ref, w1_ref, w2_ref, mult_ref, out_ref):
    out_ref[0] = _mlp_common(x_ref, n_ref, w1_ref, w2_ref) * mult_ref[0]


def _mlp_call(latent, n2, w1_t, w2_t, mult=None):
    in_specs = [
        pl.BlockSpec((1, CAP, DIM), lambda b: (b, 0, 0)),
        pl.BlockSpec((1, DIM), lambda b: (0, 0)),
        pl.BlockSpec((DIM, 2 * DIM), lambda b: (0, 0)),
        pl.BlockSpec((DIM, DIM), lambda b: (0, 0)),
    ]
    args = [latent, n2, w1_t, w2_t]
    body = _mlp_body
    if mult is not None:
        in_specs.append(pl.BlockSpec((1, CAP, 1), lambda b: (b, 0, 0)))
        args.append(mult)
        body = _mlp_final_body
    return pl.pallas_call(
        body,
        grid=(BATCH,),
        in_specs=in_specs,
        out_specs=pl.BlockSpec((1, CAP, DIM), lambda b: (b, 0, 0)),
        out_shape=jax.ShapeDtypeStruct((BATCH, CAP, DIM), jnp.float32),
    )(*args)


@functools.cache
def _sc_mesh():
    return plsc.VectorSubcoreMesh(core_axis_name="c", subcore_axis_name="s",
                                  num_cores=SC_NC, num_subcores=SC_NS)


@functools.cache
def _sc_gather_kernel():
    def body(x_hbm, idx_hbm, out_hbm, idx_v, rows_v, sem):
        wid = lax.axis_index("s") * SC_NC + lax.axis_index("c")
        base = wid * G_ROWS
        pltpu.sync_copy(idx_hbm.at[pl.ds(base, G_ROWS)], idx_v)
        pltpu.async_copy(x_hbm.at[idx_v], rows_v, sem).wait()
        pltpu.sync_copy(rows_v, out_hbm.at[pl.ds(base, G_ROWS)])

    return pl.kernel(
        body,
        out_type=jax.ShapeDtypeStruct((BATCH * CAP, DIM), jnp.float32),
        mesh=_sc_mesh(),
        scratch_types=[
            pltpu.VMEM((G_ROWS,), jnp.int32),
            pltpu.VMEM((G_ROWS, DIM), jnp.float32),
            pltpu.SemaphoreType.DMA,
        ],
    )


def _sc_gather(x2d, gidx):
    return _sc_gather_kernel()(x2d, gidx)


@functools.cache
def _sc_scatter_kernel():
    def body(lat_hbm, gidx_hbm, anti_hbm, zero_hbm, out_hbm,
             idx_v, rows_v, sem):
        wid = lax.axis_index("s") * SC_NC + lax.axis_index("c")
        base = wid * G_ROWS
        pltpu.sync_copy(gidx_hbm.at[pl.ds(base, G_ROWS)], idx_v)
        pltpu.sync_copy(lat_hbm.at[pl.ds(base, G_ROWS)], rows_v)
        pltpu.async_copy(rows_v, out_hbm.at[idx_v], sem).wait()
        pltpu.sync_copy(anti_hbm.at[pl.ds(base, G_ROWS)], idx_v)
        pltpu.sync_copy(zero_hbm, rows_v)
        pltpu.async_copy(rows_v, out_hbm.at[idx_v], sem).wait()

    return pl.kernel(
        body,
        out_type=jax.ShapeDtypeStruct((BATCH * SEQ, DIM), jnp.float32),
        mesh=_sc_mesh(),
        scratch_types=[
            pltpu.VMEM((G_ROWS,), jnp.int32),
            pltpu.VMEM((G_ROWS, DIM), jnp.float32),
            pltpu.SemaphoreType.DMA,
        ],
    )


def _sc_scatter(lat, gidx, anti, zero_rows):
    return _sc_scatter_kernel()(lat, gidx, anti, zero_rows)


def kernel(x, norm1_w, norm2_w, W_qkv, W_proj, W_fc1, W_fc2, W_router):
    f32 = jnp.float32

    i = jnp.arange(CAP, dtype=jnp.int32)[:, None]
    j = jnp.arange(CAP, dtype=jnp.int32)[None, :]
    e_mat = jnp.where(j <= i, (j - i).astype(f32), jnp.float32(-1e30))
    slopes = jnp.exp2(-((jnp.arange(QH, dtype=f32) + 1) * 8.0 / QH))

    rw_cols3 = jax.nn.sigmoid(jnp.einsum('bsd,od->bso', x, W_router))
    rw_rows3 = rw_cols3.reshape(BATCH, 1, SEQ)
    dec3, gidx3, anti3, mult3 = _select_call(rw_rows3, rw_cols3)
    gidx = gidx3.reshape(BATCH * CAP)
    anti = anti3.reshape(BATCH * CAP)

    x2d = x.reshape(BATCH * SEQ, DIM)
    latent = _sc_gather(x2d, gidx).reshape(BATCH, CAP, DIM)

    for blk in range(NBLK):
        wqkv_t = jnp.transpose(W_qkv[blk])
        wproj_t = jnp.transpose(W_proj[blk])
        w1_t = jnp.transpose(W_fc1[blk])
        w2_t = jnp.transpose(W_fc2[blk])
        n1 = norm1_w[blk].reshape(1, DIM)
        n2 = norm2_w[blk].reshape(1, DIM)
        qkv = _qkv_call(latent, n1, wqkv_t)
        attn = _att_call(qkv, e_mat, slopes)
        latent = _proj_call(attn, wproj_t, latent)
        latent = _mlp_call(latent, n2, w1_t, w2_t,
                           mult=mult3 if blk == NBLK - 1 else None)

    zero_rows = jnp.zeros((G_ROWS, DIM), f32)
    pred = _sc_scatter(latent.reshape(BATCH * CAP, DIM), gidx, anti,
                       zero_rows)
    pred = pred.reshape(BATCH, SEQ, DIM)

    return pred, rw_cols3, dec3.reshape(BATCH, SEQ, 1)


# speedup vs baseline: 3.8418x; 1.2602x over previous
import functools

import jax
import jax.numpy as jnp
from jax import lax
from jax.experimental import pallas as pl
from jax.experimental.pallas import tpu as pltpu
from jax.experimental.pallas import tpu_sc as plsc

NBLK = 2
DIM = 1024
QH = 16
KVH = 4
NH = QH + 2 * KVH
HD = 64
BATCH = 2
SEQ = 2048
CAP = 1024
CHUNK = 256
NCH = SEQ // CHUNK
SCALE = 1.0 / (HD ** 0.5)
EPS = 1e-6

_PREC = lax.Precision.DEFAULT

SC_NC = 2
SC_NS = 16
SC_NW = SC_NC * SC_NS
G_ROWS = (BATCH * CAP) // SC_NW


def _select_body(vrow_ref, vcol_ref, dec_ref, gidx_ref, anti_ref, mult_ref):
    v_row = vrow_ref[0]
    v_col = vcol_ref[0]

    rank_row = jnp.zeros((1, SEQ), jnp.int32)
    for c in range(NCH):
        vs = v_col[c * CHUNK:(c + 1) * CHUNK, :]
        s_idx = lax.broadcasted_iota(jnp.int32, (CHUNK, SEQ), 0) + c * CHUNK
        t_idx = lax.broadcasted_iota(jnp.int32, (CHUNK, SEQ), 1)
        beat = (vs > v_row) | ((vs == v_row) & (s_idx < t_idx))
        rank_row += jnp.sum(beat.astype(jnp.int32), axis=0, keepdims=True)
    sel_row = rank_row < CAP
    dec_ref[0] = sel_row.astype(jnp.float32)

    rank_col = jnp.zeros((SEQ, 1), jnp.int32)
    for c in range(NCH):
        vt = v_row[:, c * CHUNK:(c + 1) * CHUNK]
        t_idx = lax.broadcasted_iota(jnp.int32, (SEQ, CHUNK), 1) + c * CHUNK
        s_idx = lax.broadcasted_iota(jnp.int32, (SEQ, CHUNK), 0)
        beat = (vt > v_col) | ((vt == v_col) & (t_idx < s_idx))
        rank_col += jnp.sum(beat.astype(jnp.int32), axis=1, keepdims=True)
    sel_col = rank_col < CAP

    pos_row = jnp.zeros((1, SEQ), jnp.int32)
    for c in range(NCH):
        selc = sel_col[c * CHUNK:(c + 1) * CHUNK, :]
        s_idx = lax.broadcasted_iota(jnp.int32, (CHUNK, SEQ), 0) + c * CHUNK
        t_idx = lax.broadcasted_iota(jnp.int32, (CHUNK, SEQ), 1)
        pos_row += jnp.sum((selc & (s_idx < t_idx)).astype(jnp.int32),
                           axis=0, keepdims=True)

    w_row = jnp.zeros((1, SEQ), jnp.float32)
    for c in range(NCH):
        vr = v_col[c * CHUNK:(c + 1) * CHUNK, :]
        r_idx = lax.broadcasted_iota(jnp.int32, (CHUNK, SEQ), 0) + c * CHUNK
        w_row += jnp.sum(jnp.where(r_idx == rank_row, vr, 0.0),
                         axis=0, keepdims=True)

    jv = lax.broadcasted_iota(jnp.int32, (CAP, 1), 0)
    sidx_acc = jnp.zeros((CAP, 1), jnp.int32)
    anti_acc = jnp.zeros((CAP, 1), jnp.int32)
    mult_acc = jnp.zeros((CAP, 1), jnp.float32)
    for c in range(NCH):
        selr = sel_row[:, c * CHUNK:(c + 1) * CHUNK]
        posr = pos_row[:, c * CHUNK:(c + 1) * CHUNK]
        wr = w_row[:, c * CHUNK:(c + 1) * CHUNK]
        s_iota = lax.broadcasted_iota(jnp.int32, (1, CHUNK), 1) + c * CHUNK
        m2 = selr & (posr == jv)
        sidx_acc += jnp.sum(jnp.where(m2, s_iota, 0), axis=1, keepdims=True)
        mult_acc += jnp.sum(jnp.where(m2, wr, 0.0), axis=1, keepdims=True)
        antipos = s_iota - posr
        m3 = (~selr) & (antipos == jv)
        anti_acc += jnp.sum(jnp.where(m3, s_iota, 0), axis=1, keepdims=True)

    b = pl.program_id(0)
    gidx_ref[0] = sidx_acc + b * SEQ
    anti_ref[0] = anti_acc + b * SEQ
    mult_ref[0] = mult_acc


def _select_call(rw_rows3, rw_cols3):
    return pl.pallas_call(
        _select_body,
        grid=(BATCH,),
        in_specs=[
            pl.BlockSpec((1, 1, SEQ), lambda b: (b, 0, 0)),
            pl.BlockSpec((1, SEQ, 1), lambda b: (b, 0, 0)),
        ],
        out_specs=[
            pl.BlockSpec((1, 1, SEQ), lambda b: (b, 0, 0)),
            pl.BlockSpec((1, CAP, 1), lambda b: (b, 0, 0)),
            pl.BlockSpec((1, CAP, 1), lambda b: (b, 0, 0)),
            pl.BlockSpec((1, CAP, 1), lambda b: (b, 0, 0)),
        ],
        out_shape=[
            jax.ShapeDtypeStruct((BATCH, 1, SEQ), jnp.float32),
            jax.ShapeDtypeStruct((BATCH, CAP, 1), jnp.int32),
            jax.ShapeDtypeStruct((BATCH, CAP, 1), jnp.int32),
            jax.ShapeDtypeStruct((BATCH, CAP, 1), jnp.float32),
        ],
    )(rw_rows3, rw_cols3)


_SLOPES = [2.0 ** (-(h + 1) * 8.0 / QH) for h in range(QH)]


QHALF = CAP // 2


def _alibi_bias(rows, cols, row0):
    i = lax.broadcasted_iota(jnp.int32, (rows, cols), 0) + row0
    j = lax.broadcasted_iota(jnp.int32, (rows, cols), 1)
    return jnp.where(j <= i, (j - i).astype(jnp.float32),
                     jnp.float32(-1e30))


def _attblk_body(x_ref, n_ref, wqkv_ref, wp_ref, out_ref, a2_scr):
    xb = x_ref[0]
    var = jnp.mean(xb * xb, axis=1, keepdims=True)
    hn = xb * lax.rsqrt(var + EPS) * n_ref[...]
    qkv = jnp.dot(hn, wqkv_ref[...], preferred_element_type=jnp.float32,
                  precision=_PREC)
    e0 = _alibi_bias(QHALF, QHALF, 0)
    e1 = _alibi_bias(QHALF, CAP, QHALF)
    for h in range(QH):
        g = h // 4
        slope = jnp.float32(_SLOPES[h])
        k = qkv[:, (QH + g) * HD:(QH + g + 1) * HD]
        v = qkv[:, (QH + KVH + g) * HD:(QH + KVH + g + 1) * HD]
        for qh in range(2):
            q = qkv[qh * QHALF:(qh + 1) * QHALF, h * HD:(h + 1) * HD]
            kk = k[:QHALF] if qh == 0 else k
            vv = v[:QHALF] if qh == 0 else v
            e = e0 if qh == 0 else e1
            scores = lax.dot_general(q, kk, (((1,), (1,)), ((), ())),
                                     preferred_element_type=jnp.float32,
                                     precision=_PREC)
            s2 = scores * SCALE + slope * e
            m = jnp.max(s2, axis=1, keepdims=True)
            p = jnp.exp(s2 - m)
            l = jnp.sum(p, axis=1, keepdims=True)
            o = jnp.dot(p, vv, preferred_element_type=jnp.float32,
                        precision=_PREC)
            a2_scr[qh * QHALF:(qh + 1) * QHALF,
                   h * HD:(h + 1) * HD] = o / l
    out_ref[0] = xb + jnp.dot(a2_scr[...], wp_ref[...],
                              preferred_element_type=jnp.float32,
                              precision=_PREC)


def _attblk_call(latent, n1, wqkv_t, wproj_t):
    return pl.pallas_call(
        _attblk_body,
        grid=(BATCH,),
        in_specs=[
            pl.BlockSpec((1, CAP, DIM), lambda b: (b, 0, 0)),
            pl.BlockSpec((1, DIM), lambda b: (0, 0)),
            pl.BlockSpec((DIM, NH * HD), lambda b: (0, 0)),
            pl.BlockSpec((DIM, DIM), lambda b: (0, 0)),
        ],
        out_specs=pl.BlockSpec((1, CAP, DIM), lambda b: (b, 0, 0)),
        out_shape=jax.ShapeDtypeStruct((BATCH, CAP, DIM), jnp.float32),
        scratch_shapes=[pltpu.VMEM((CAP, DIM), jnp.float32)],
    )(latent, n1, wqkv_t, wproj_t)


def _mlp_common(x_ref, n_ref, w1_ref, w2_ref):
    xb = x_ref[0]
    var = jnp.mean(xb * xb, axis=1, keepdims=True)
    h = xb * lax.rsqrt(var + EPS) * n_ref[...]
    hh = jnp.dot(h, w1_ref[...], preferred_element_type=jnp.float32,
                 precision=_PREC)
    x1 = hh[:, :DIM]
    gate = hh[:, DIM:]
    act = x1 * (gate * (1.0 / (1.0 + jnp.exp(-gate))))
    y = jnp.dot(act, w2_ref[...], preferred_element_type=jnp.float32,
                precision=_PREC)
    return xb + y


def _mlp_body(x_ref, n_ref, w1_ref, w2_ref, out_ref):
    out_ref[0] = _mlp_common(x_ref, n_ref, w1_ref, w2_ref)


def _mlp_final_body(x_ref, n_ref, w1_ref, w2_ref, mult_ref, out_ref):
    out_ref[0] = _mlp_common(x_ref, n_ref, w1_ref, w2_ref) * mult_ref[0]


def _mlp_call(latent, n2, w1_t, w2_t, mult=None):
    in_specs = [
        pl.BlockSpec((1, CAP, DIM), lambda b: (b, 0, 0)),
        pl.BlockSpec((1, DIM), lambda b: (0, 0)),
        pl.BlockSpec((DIM, 2 * DIM), lambda b: (0, 0)),
        pl.BlockSpec((DIM, DIM), lambda b: (0, 0)),
    ]
    args = [latent, n2, w1_t, w2_t]
    body = _mlp_body
    if mult is not None:
        in_specs.append(pl.BlockSpec((1, CAP, 1), lambda b: (b, 0, 0)))
        args.append(mult)
        body = _mlp_final_body
    return pl.pallas_call(
        body,
        grid=(BATCH,),
        in_specs=in_specs,
        out_specs=pl.BlockSpec((1, CAP, DIM), lambda b: (b, 0, 0)),
        out_shape=jax.ShapeDtypeStruct((BATCH, CAP, DIM), jnp.float32),
    )(*args)


@functools.cache
def _sc_mesh():
    return plsc.VectorSubcoreMesh(core_axis_name="c", subcore_axis_name="s",
                                  num_cores=SC_NC, num_subcores=SC_NS)


@functools.cache
def _sc_gather_kernel():
    def body(x_hbm, idx_hbm, out_hbm, idx_v, rows_v, sem):
        wid = lax.axis_index("s") * SC_NC + lax.axis_index("c")
        base = wid * G_ROWS
        pltpu.sync_copy(idx_hbm.at[pl.ds(base, G_ROWS)], idx_v)
        pltpu.async_copy(x_hbm.at[idx_v], rows_v, sem).wait()
        pltpu.sync_copy(rows_v, out_hbm.at[pl.ds(base, G_ROWS)])

    return pl.kernel(
        body,
        out_type=jax.ShapeDtypeStruct((BATCH * CAP, DIM), jnp.float32),
        mesh=_sc_mesh(),
        scratch_types=[
            pltpu.VMEM((G_ROWS,), jnp.int32),
            pltpu.VMEM((G_ROWS, DIM), jnp.float32),
            pltpu.SemaphoreType.DMA,
        ],
    )


def _sc_gather(x2d, gidx):
    return _sc_gather_kernel()(x2d, gidx)


@functools.cache
def _sc_scatter_kernel():
    def body(lat_hbm, gidx_hbm, anti_hbm, zero_hbm, out_hbm,
             idx_v, rows_v, sem):
        wid = lax.axis_index("s") * SC_NC + lax.axis_index("c")
        base = wid * G_ROWS
        pltpu.sync_copy(gidx_hbm.at[pl.ds(base, G_ROWS)], idx_v)
        pltpu.sync_copy(lat_hbm.at[pl.ds(base, G_ROWS)], rows_v)
        pltpu.async_copy(rows_v, out_hbm.at[idx_v], sem).wait()
        pltpu.sync_copy(anti_hbm.at[pl.ds(base, G_ROWS)], idx_v)
        pltpu.sync_copy(zero_hbm, rows_v)
        pltpu.async_copy(rows_v, out_hbm.at[idx_v], sem).wait()

    return pl.kernel(
        body,
        out_type=jax.ShapeDtypeStruct((BATCH * SEQ, DIM), jnp.float32),
        mesh=_sc_mesh(),
        scratch_types=[
            pltpu.VMEM((G_ROWS,), jnp.int32),
            pltpu.VMEM((G_ROWS, DIM), jnp.float32),
            pltpu.SemaphoreType.DMA,
        ],
    )


def _sc_scatter(lat, gidx, anti, zero_rows):
    return _sc_scatter_kernel()(lat, gidx, anti, zero_rows)


def kernel(x, norm1_w, norm2_w, W_qkv, W_proj, W_fc1, W_fc2, W_router):
    f32 = jnp.float32

    rw_cols3 = jax.nn.sigmoid(jnp.einsum('bsd,od->bso', x, W_router))
    rw_rows3 = rw_cols3.reshape(BATCH, 1, SEQ)
    dec3, gidx3, anti3, mult3 = _select_call(rw_rows3, rw_cols3)
    gidx = gidx3.reshape(BATCH * CAP)
    anti = anti3.reshape(BATCH * CAP)

    x2d = x.reshape(BATCH * SEQ, DIM)
    latent = _sc_gather(x2d, gidx).reshape(BATCH, CAP, DIM)

    for blk in range(NBLK):
        wqkv_t = jnp.transpose(W_qkv[blk])
        wproj_t = jnp.transpose(W_proj[blk])
        w1_t = jnp.transpose(W_fc1[blk])
        w2_t = jnp.transpose(W_fc2[blk])
        n1 = norm1_w[blk].reshape(1, DIM)
        n2 = norm2_w[blk].reshape(1, DIM)
        latent = _attblk_call(latent, n1, wqkv_t, wproj_t)
        latent = _mlp_call(latent, n2, w1_t, w2_t,
                           mult=mult3 if blk == NBLK - 1 else None)

    zero_rows = jnp.zeros((G_ROWS, DIM), f32)
    pred = _sc_scatter(latent.reshape(BATCH * CAP, DIM), gidx, anti,
                       zero_rows)
    pred = pred.reshape(BATCH, SEQ, DIM)

    return pred, rw_cols3, dec3.reshape(BATCH, SEQ, 1)
